# Initial kernel scaffold; baseline (speedup 1.0000x reference)
import jax
import jax.numpy as jnp
from jax.experimental import pallas as pl


def kernel(x, p_grid):
    raise NotImplementedError("write your pallas kernel here")


# speedup vs baseline: 9.2334x; 9.2334x over previous
import jax
import jax.numpy as jnp
from jax import lax
from jax.experimental import pallas as pl

_RADIUS2 = 0.25 * 0.25
_K = 10
_NK = 32 * 16 * 16
_BQ = 256
_NQPAD = 10240


def _bq_kernel(x_ref, gt_ref, map_ref, out_ref):
    xb = x_ref[...]
    gt = gt_ref[...]
    qn = jnp.sum(xb * xb, axis=1, keepdims=True)
    kn = jnp.sum(gt * gt, axis=0, keepdims=True)
    cross = lax.dot_general(
        xb, gt, (((1,), (0,)), ((), ())),
        precision=lax.Precision.DEFAULT,
        preferred_element_type=jnp.float32)
    d2 = (qn + kn) - 2.0 * cross
    s = jnp.where(d2 <= _RADIUS2, -d2, -jnp.inf)
    iota = lax.broadcasted_iota(jnp.int32, (_BQ, _NK), 1)
    maps = []
    coords = []
    for _ in range(_K):
        m = jnp.max(s, axis=1, keepdims=True)
        valid = m > -jnp.inf
        eq = s == m
        cand = jnp.where(eq, iota, _NK)
        idx = jnp.min(cand, axis=1, keepdims=True)
        maps.append(jnp.where(valid, idx, 0))
        oh = iota == idx
        ohf = jnp.where(oh & valid, 1.0, 0.0).astype(jnp.float32)
        coords.append(lax.dot_general(
            ohf, gt, (((1,), (1,)), ((), ())),
            precision=lax.Precision.HIGHEST,
            preferred_element_type=jnp.float32))
        s = jnp.where(oh, -jnp.inf, s)
    map_ref[...] = jnp.concatenate(maps, axis=1)
    out_ref[...] = jnp.concatenate(coords, axis=1)


def _run(xp, gt, interpret=False):
    nblk = _NQPAD // _BQ
    return pl.pallas_call(
        _bq_kernel,
        grid=(nblk,),
        in_specs=[pl.BlockSpec((_BQ, 3), lambda i: (i, 0)),
                  pl.BlockSpec((3, _NK), lambda i: (0, 0))],
        out_specs=[pl.BlockSpec((_BQ, _K), lambda i: (i, 0)),
                   pl.BlockSpec((_BQ, 3 * _K), lambda i: (i, 0))],
        out_shape=[jax.ShapeDtypeStruct((_NQPAD, _K), jnp.int32),
                   jax.ShapeDtypeStruct((_NQPAD, 3 * _K), jnp.float32)],
        interpret=interpret,
    )(xp, gt)


def kernel(x, p_grid):
    b, nq, _ = x.shape
    gt = jnp.reshape(p_grid, (-1, 3)).T
    xq = jnp.reshape(x, (nq, 3))
    xp = jnp.concatenate(
        [xq, jnp.full((_NQPAD - nq, 3), 2.0, jnp.float32)], axis=0)
    mp, co = _run(xp, gt)
    mapping = mp[:nq].astype(jnp.int64).reshape(b, nq, _K)
    outputs = co[:nq].reshape(b, nq, _K, 3)
    return (mapping, outputs)


# speedup vs baseline: 9.2515x; 1.0020x over previous
import jax
import jax.numpy as jnp
from jax import lax
from jax.experimental import pallas as pl

_RADIUS2 = 0.25 * 0.25
_K = 10
_NK = 32 * 16 * 16
_BQ = 256
_NQPAD = 10240


def _bq_kernel(x_ref, gt_ref, map_ref, out_ref):
    xb = x_ref[...]
    gt = gt_ref[...]
    x0, x1, x2 = xb[:, 0:1], xb[:, 1:2], xb[:, 2:3]
    g0, g1, g2 = gt[0:1, :], gt[1:2, :], gt[2:3, :]
    qn = (x0 * x0 + x1 * x1) + x2 * x2
    kn = (g0 * g0 + g1 * g1) + g2 * g2
    cross = lax.dot_general(
        xb, gt, (((1,), (0,)), ((), ())),
        precision=lax.Precision.DEFAULT,
        preferred_element_type=jnp.float32)
    d2 = (qn + kn) - 2.0 * cross
    s = jnp.where(d2 <= _RADIUS2, -d2, -jnp.inf)
    iota = lax.broadcasted_iota(jnp.int32, (_BQ, _NK), 1)
    maps = []
    coords = []
    for _ in range(_K):
        m = jnp.max(s, axis=1, keepdims=True)
        valid = m > -jnp.inf
        eq = s == m
        cand = jnp.where(eq, iota, _NK)
        idx = jnp.min(cand, axis=1, keepdims=True)
        maps.append(jnp.where(valid, idx, 0))
        oh = iota == idx
        ohf = jnp.where(oh & valid, 1.0, 0.0).astype(jnp.float32)
        coords.append(lax.dot_general(
            ohf, gt, (((1,), (1,)), ((), ())),
            precision=lax.Precision.HIGHEST,
            preferred_element_type=jnp.float32))
        s = jnp.where(oh, -jnp.inf, s)
    map_ref[...] = jnp.concatenate(maps, axis=1)
    out_ref[...] = jnp.concatenate(coords, axis=1)


def _run(xp, gt, interpret=False):
    nblk = _NQPAD // _BQ
    return pl.pallas_call(
        _bq_kernel,
        grid=(nblk,),
        in_specs=[pl.BlockSpec((_BQ, 3), lambda i: (i, 0)),
                  pl.BlockSpec((3, _NK), lambda i: (0, 0))],
        out_specs=[pl.BlockSpec((_BQ, _K), lambda i: (i, 0)),
                   pl.BlockSpec((_BQ, 3 * _K), lambda i: (i, 0))],
        out_shape=[jax.ShapeDtypeStruct((_NQPAD, _K), jnp.int32),
                   jax.ShapeDtypeStruct((_NQPAD, 3 * _K), jnp.float32)],
        interpret=interpret,
    )(xp, gt)


def kernel(x, p_grid):
    b, nq, _ = x.shape
    gt = jnp.reshape(p_grid, (-1, 3)).T
    xq = jnp.reshape(x, (nq, 3))
    xp = jnp.concatenate(
        [xq, jnp.full((_NQPAD - nq, 3), 2.0, jnp.float32)], axis=0)
    mp, co = _run(xp, gt)
    mapping = mp[:nq].astype(jnp.int64).reshape(b, nq, _K)
    outputs = co[:nq].reshape(b, nq, _K, 3)
    return (mapping, outputs)


# speedup vs baseline: 34.0898x; 3.6848x over previous
import functools

import jax
import jax.numpy as jnp
from jax import lax
from jax.experimental import pallas as pl
from jax.experimental.pallas import tpu as pltpu
from jax.experimental.pallas import tpu_sc as plsc

_RADIUS2 = 0.25 * 0.25
_K = 10
_NK = 32 * 16 * 16
_BQ = 256
_NQPAD = 10240

_D = 16
_NW = 32
_B_IDS = _NQPAD * _K
_B_PER_W = _B_IDS // _NW


def _bq_kernel(x_ref, gt_ref, map_ref, ids_ref):
    xb = x_ref[...]
    gt = gt_ref[...]
    x0, x1, x2 = xb[:, 0:1], xb[:, 1:2], xb[:, 2:3]
    g0, g1, g2 = gt[0:1, :], gt[1:2, :], gt[2:3, :]
    qn = (x0 * x0 + x1 * x1) + x2 * x2
    kn = (g0 * g0 + g1 * g1) + g2 * g2
    cross = lax.dot_general(
        xb, gt, (((1,), (0,)), ((), ())),
        precision=lax.Precision.DEFAULT,
        preferred_element_type=jnp.float32)
    d2 = (qn + kn) - 2.0 * cross
    s = jnp.where(d2 <= _RADIUS2, -d2, -jnp.inf)
    iota = lax.broadcasted_iota(jnp.int32, (_BQ, _NK), 1)
    maps = []
    ids = []
    for _ in range(_K):
        m = jnp.max(s, axis=1, keepdims=True)
        valid = m > -jnp.inf
        eq = s == m
        cand = jnp.where(eq, iota, _NK)
        idx = jnp.min(cand, axis=1, keepdims=True)
        maps.append(jnp.where(valid, idx, 0))
        ids.append(jnp.where(valid, idx, _NK))
        s = jnp.where(iota == idx, -jnp.inf, s)
    map_ref[...] = jnp.concatenate(maps, axis=1)
    ids_ref[...] = jnp.concatenate(ids, axis=1)


def _run_tc(xp, gt, interpret=False):
    nblk = _NQPAD // _BQ
    return pl.pallas_call(
        _bq_kernel,
        grid=(nblk,),
        in_specs=[pl.BlockSpec((_BQ, 3), lambda i: (i, 0)),
                  pl.BlockSpec((3, _NK), lambda i: (0, 0))],
        out_specs=[pl.BlockSpec((_BQ, _K), lambda i: (i, 0)),
                   pl.BlockSpec((_BQ, _K), lambda i: (i, 0))],
        out_shape=[jax.ShapeDtypeStruct((_NQPAD, _K), jnp.int32),
                   jax.ShapeDtypeStruct((_NQPAD, _K), jnp.int32)],
        interpret=interpret,
    )(xp, gt)


@functools.partial(
    pl.kernel,
    mesh=plsc.VectorSubcoreMesh(core_axis_name="c", subcore_axis_name="s"),
    out_type=jax.ShapeDtypeStruct((_B_IDS, _D), jnp.float32),
    scratch_types=[pltpu.VMEM((_B_PER_W,), jnp.int32),
                   pltpu.VMEM((_B_PER_W, _D), jnp.float32),
                   pltpu.SemaphoreType.DMA],
    compiler_params=pltpu.CompilerParams(use_tc_tiling_on_sc=False),
)
def _sc_gather(table_hbm, idx_hbm, out_hbm, idx_v, rows_v, sem):
    wid = lax.axis_index("s") * 2 + lax.axis_index("c")
    base = wid * _B_PER_W
    pltpu.sync_copy(idx_hbm.at[pl.ds(base, _B_PER_W)], idx_v)
    pltpu.async_copy(table_hbm.at[idx_v], rows_v, sem).wait()
    pltpu.sync_copy(rows_v, out_hbm.at[pl.ds(base, _B_PER_W)])


def kernel(x, p_grid):
    b, nq, _ = x.shape
    grid_flat = jnp.reshape(p_grid, (-1, 3))
    gt = grid_flat.T
    xq = jnp.reshape(x, (nq, 3))
    xp = jnp.concatenate(
        [xq, jnp.full((_NQPAD - nq, 3), 2.0, jnp.float32)], axis=0)
    mp, ids = _run_tc(xp, gt)
    table = jnp.pad(grid_flat, ((0, 1), (0, _D - 3)))
    rows = _sc_gather(table, jnp.reshape(ids, (_B_IDS,)))
    mapping = mp[:nq].astype(jnp.int64).reshape(b, nq, _K)
    outputs = jnp.reshape(rows, (_NQPAD, _K, _D))[:nq, :, :3].reshape(
        b, nq, _K, 3)
    return (mapping, outputs)
